# baseline (speedup 1.0000x reference)
import functools

import jax
import jax.numpy as jnp
import numpy as np
from jax import lax
from jax.experimental import pallas as pl
from jax.experimental.pallas import tpu as pltpu
from jax.experimental.pallas import tpu_sc as plsc

_B = 16384
_F = 512
_BLK_ROWS = 512

_SC_ROWS = 4096
_NW = 32
_SC_ELEMS_W = _SC_ROWS * _F // _NW
_SC_CHUNKS = _SC_ELEMS_W // 16

_KS0 = np.uint32(0)
_KS1 = np.uint32(1)
_KS2 = np.uint32(0x1BD11BDA) ^ _KS0 ^ _KS1
_KS = (_KS0, _KS1, _KS2)
_ROTATIONS = ((13, 15, 26, 6), (17, 29, 16, 24))

_LO = np.float32(np.nextafter(np.float32(-1.0), np.float32(0.0)))

_ERFINV_COEF = (
    np.float32(-0.018726321), np.float32(0.05377644),
    np.float32(0.31786564), np.float32(-0.016997397),
    np.float32(1.259041),
)

_ERFINV_W_COEF = (
    np.float32(3.9184595e-08), np.float32(-1.2624944e-06),
    np.float32(3.1340858e-06), np.float32(0.00032718584),
    np.float32(-0.004842447), np.float32(0.018431418),
    np.float32(0.32697421), np.float32(1.2534769),
)

_MLOG_COEF = (
    np.float32(-0.03010224737226963), np.float32(0.28062915802001953),
    np.float32(-1.1047965288162231), np.float32(2.420793056488037),
    np.float32(-3.4982118606567383), np.float32(89.96135711669922),
)
_NEG_LN2 = np.float32(-np.log(2.0))


def _rotl(v, r):
    return (v << np.uint32(r)) | (v >> np.uint32(32 - r))


def _threefry2x32_xored(c1):
    x1 = c1
    x0 = x1
    x1 = _rotl(x1, 13) ^ x0
    for r in (15, 26, 6):
        x0 = x0 + x1
        x1 = _rotl(x1, r)
        x1 = x0 ^ x1
    x0 = x0 + _KS[1]
    x1 = x1 + (_KS[2] + np.uint32(1))
    for i in range(1, 5):
        for r in _ROTATIONS[i % 2]:
            x0 = x0 + x1
            x1 = _rotl(x1, r)
            x1 = x0 ^ x1
        if (i + 1) % 3 != 0:
            x0 = x0 + _KS[(i + 1) % 3]
        x1 = x1 + (_KS[(i + 2) % 3] + np.uint32(i + 1))
    return x0 ^ x1


def _bits_to_normal(bits):
    fb = pltpu.bitcast((bits >> np.uint32(9)) | np.uint32(0x40000000),
                       jnp.float32)
    x = jnp.maximum(_LO, fb - np.float32(3.0))
    t = jnp.sqrt(jnp.log(np.float32(1.0) - x * x) * np.float32(-1.0))
    p = _ERFINV_COEF[0]
    for c in _ERFINV_COEF[1:]:
        p = c + p * t
    return p * x


def _noise_kernel(y_ref, mu_ref, sg_ref, o_ref):
    j = pl.program_id(0) + np.uint32(_SC_ROWS // _BLK_ROWS)
    base1 = (j * np.uint32(_BLK_ROWS * _F) + np.uint32(1)).astype(jnp.uint32)
    row = jax.lax.broadcasted_iota(jnp.uint32, (_BLK_ROWS, _F), 0)
    col = jax.lax.broadcasted_iota(jnp.uint32, (_BLK_ROWS, _F), 1)
    c1 = base1 + row * np.uint32(_F) + col
    z = _bits_to_normal(_threefry2x32_xored(c1))
    scale = jnp.exp(sg_ref[...])
    o_ref[...] = y_ref[...] + (z * scale + mu_ref[...])


def _tc_call(y, mu, sg):
    off = _SC_ROWS // _BLK_ROWS
    return pl.pallas_call(
        _noise_kernel,
        grid=((_B - _SC_ROWS) // _BLK_ROWS,),
        in_specs=[
            pl.BlockSpec((_BLK_ROWS, _F), lambda j: (j + off, 0)),
            pl.BlockSpec((1, _F), lambda j: (0, 0)),
            pl.BlockSpec((1, _F), lambda j: (0, 0)),
        ],
        out_specs=pl.BlockSpec((_BLK_ROWS, _F), lambda j: (j + off, 0)),
        out_shape=jax.ShapeDtypeStruct((_B, _F), jnp.float32),
        compiler_params=pltpu.CompilerParams(
            dimension_semantics=("parallel",),
        ),
    )(y, mu, sg)


def _rotl_i32(v, r):
    return lax.shift_left(v, np.int32(r)) | lax.shift_right_logical(
        v, np.int32(32 - r))


def _threefry_xored_i32(c1):
    ks2 = np.int32(0x1BD11BDB)
    x1 = c1
    x0 = x1
    x1 = _rotl_i32(x1, 13) ^ x0
    for r in (15, 26, 6):
        x0 = x0 + x1
        x1 = _rotl_i32(x1, r)
        x1 = x0 ^ x1
    x0 = x0 + np.int32(1)
    x1 = x1 + (ks2 + np.int32(1))
    rot_groups = (
        ((17, 29, 16, 24), ks2, np.int32(2)),
        ((13, 15, 26, 6), None, np.int32(4)),
        ((17, 29, 16, 24), np.int32(1), ks2 + np.int32(4)),
        ((13, 15, 26, 6), ks2, np.int32(5)),
    )
    for rots, k0inj, k1inj in rot_groups:
        for r in rots:
            x0 = x0 + x1
            x1 = _rotl_i32(x1, r)
            x1 = x0 ^ x1
        if k0inj is not None:
            x0 = x0 + k0inj
        x1 = x1 + k1inj
    return x0 ^ x1


def _sc_bits_to_normal(bits):
    fb = lax.bitcast_convert_type(
        lax.shift_right_logical(bits, np.int32(9)) | np.int32(0x40000000),
        jnp.float32)
    x = jnp.maximum(_LO, fb - np.float32(3.0))
    u = np.float32(1.0) - x * x
    bu = lax.bitcast_convert_type(u, jnp.int32)
    e_f = lax.convert_element_type(
        lax.shift_right_logical(bu, np.int32(23)), jnp.float32)
    mf = lax.bitcast_convert_type(
        (bu & np.int32(0x007FFFFF)) | np.int32(0x3F800000), jnp.float32)
    q = _MLOG_COEF[0]
    for c in _MLOG_COEF[1:]:
        q = c + q * mf
    w = e_f * _NEG_LN2 + q
    p = _ERFINV_W_COEF[0]
    for c in _ERFINV_W_COEF[1:]:
        p = c + p * w
    return p * x


_SC_ROWS_W = _SC_ROWS // _NW


def _sc_noise_body(y_hbm, mu_hbm, sg_hbm, out_hbm, ybuf, mu_v, scale_v):
    wid = lax.axis_index("s") * 2 + lax.axis_index("c")
    row0 = wid * np.int32(_SC_ROWS_W)
    pltpu.sync_copy(y_hbm.at[pl.ds(row0, _SC_ROWS_W)], ybuf)
    pltpu.sync_copy(mu_hbm, mu_v)
    pltpu.sync_copy(sg_hbm, scale_v)

    def exp_body(k, carry):
        sl = pl.ds(k * 16, 16)
        scale_v[sl] = jnp.exp(scale_v[sl])
        return carry

    lax.fori_loop(0, _F // 16, exp_body, np.int32(0))

    lane = lax.iota(jnp.int32, 16)
    ebase = row0 * np.int32(_F)
    ncol = np.int32(_F // 16)

    def body(i, carry):
        c1 = lane + (ebase + i * np.int32(16) + np.int32(1))
        z = _sc_bits_to_normal(_threefry_xored_i32(c1))
        r = lax.div(i, ncol)
        col0 = lax.rem(i, ncol) * np.int32(16)
        csl = pl.ds(col0, 16)
        ybuf[r, csl] = ybuf[r, csl] + (z * scale_v[csl] + mu_v[csl])
        return carry

    lax.fori_loop(0, _SC_CHUNKS, body, np.int32(0), unroll=2)
    pltpu.sync_copy(ybuf, out_hbm.at[pl.ds(row0, _SC_ROWS_W)])


def _sc_call(y, mu_flat, sg_flat):
    mesh = plsc.VectorSubcoreMesh(core_axis_name="c", subcore_axis_name="s")
    run = functools.partial(
        pl.kernel,
        out_type=jax.ShapeDtypeStruct((_SC_ROWS, _F), jnp.float32),
        mesh=mesh,
        scratch_types=[
            pltpu.VMEM((_SC_ROWS_W, _F), jnp.float32),
            pltpu.VMEM((_F,), jnp.float32),
            pltpu.VMEM((_F,), jnp.float32),
        ],
    )(_sc_noise_body)
    return run(y, mu_flat, sg_flat)


def kernel(x, y, mean, sigma):
    del x
    out_sc = _sc_call(y, mean.reshape(_F), sigma.reshape(_F))
    out_tc = _tc_call(y, mean.reshape(1, _F), sigma.reshape(1, _F))
    return lax.dynamic_update_slice(out_tc, out_sc, (0, 0))


# speedup vs baseline: 1.0939x; 1.0939x over previous
import functools

import jax
import jax.numpy as jnp
import numpy as np
from jax import lax
from jax.experimental import pallas as pl
from jax.experimental.pallas import tpu as pltpu
from jax.experimental.pallas import tpu_sc as plsc

_B = 16384
_F = 512
_BLK_ROWS = 512

_SC_ROWS = 3584
_NW = 32
_SC_ELEMS_W = _SC_ROWS * _F // _NW
_SC_CHUNKS = _SC_ELEMS_W // 16

_KS0 = np.uint32(0)
_KS1 = np.uint32(1)
_KS2 = np.uint32(0x1BD11BDA) ^ _KS0 ^ _KS1
_KS = (_KS0, _KS1, _KS2)
_ROTATIONS = ((13, 15, 26, 6), (17, 29, 16, 24))

_LO = np.float32(np.nextafter(np.float32(-1.0), np.float32(0.0)))

_ERFINV_COEF = (
    np.float32(-0.018726321), np.float32(0.05377644),
    np.float32(0.31786564), np.float32(-0.016997397),
    np.float32(1.259041),
)

_ERFINV_W_COEF = (
    np.float32(3.9184595e-08), np.float32(-1.2624944e-06),
    np.float32(3.1340858e-06), np.float32(0.00032718584),
    np.float32(-0.004842447), np.float32(0.018431418),
    np.float32(0.32697421), np.float32(1.2534769),
)

_MLOG_COEF = (
    np.float32(-0.03010224737226963), np.float32(0.28062915802001953),
    np.float32(-1.1047965288162231), np.float32(2.420793056488037),
    np.float32(-3.4982118606567383), np.float32(89.96135711669922),
)
_NEG_LN2 = np.float32(-np.log(2.0))


def _rotl(v, r):
    return (v << np.uint32(r)) | (v >> np.uint32(32 - r))


def _threefry2x32_xored(c1):
    x1 = c1
    x0 = x1
    x1 = _rotl(x1, 13) ^ x0
    for r in (15, 26, 6):
        x0 = x0 + x1
        x1 = _rotl(x1, r)
        x1 = x0 ^ x1
    x0 = x0 + _KS[1]
    x1 = x1 + (_KS[2] + np.uint32(1))
    for i in range(1, 5):
        for r in _ROTATIONS[i % 2]:
            x0 = x0 + x1
            x1 = _rotl(x1, r)
            x1 = x0 ^ x1
        if (i + 1) % 3 != 0:
            x0 = x0 + _KS[(i + 1) % 3]
        x1 = x1 + (_KS[(i + 2) % 3] + np.uint32(i + 1))
    return x0 ^ x1


def _bits_to_normal(bits):
    fb = pltpu.bitcast((bits >> np.uint32(9)) | np.uint32(0x40000000),
                       jnp.float32)
    x = jnp.maximum(_LO, fb - np.float32(3.0))
    t = jnp.sqrt(jnp.log(np.float32(1.0) - x * x) * np.float32(-1.0))
    p = _ERFINV_COEF[0]
    for c in _ERFINV_COEF[1:]:
        p = c + p * t
    return p * x


def _noise_kernel(y_ref, mu_ref, sg_ref, o_ref):
    j = pl.program_id(0) + np.uint32(_SC_ROWS // _BLK_ROWS)
    base1 = (j * np.uint32(_BLK_ROWS * _F) + np.uint32(1)).astype(jnp.uint32)
    row = jax.lax.broadcasted_iota(jnp.uint32, (_BLK_ROWS, _F), 0)
    col = jax.lax.broadcasted_iota(jnp.uint32, (_BLK_ROWS, _F), 1)
    c1 = base1 + row * np.uint32(_F) + col
    z = _bits_to_normal(_threefry2x32_xored(c1))
    scale = jnp.exp(sg_ref[...])
    o_ref[...] = y_ref[...] + (z * scale + mu_ref[...])


def _tc_call(y, mu, sg):
    off = _SC_ROWS // _BLK_ROWS
    return pl.pallas_call(
        _noise_kernel,
        grid=((_B - _SC_ROWS) // _BLK_ROWS,),
        in_specs=[
            pl.BlockSpec((_BLK_ROWS, _F), lambda j: (j + off, 0)),
            pl.BlockSpec((1, _F), lambda j: (0, 0)),
            pl.BlockSpec((1, _F), lambda j: (0, 0)),
        ],
        out_specs=pl.BlockSpec((_BLK_ROWS, _F), lambda j: (j + off, 0)),
        out_shape=jax.ShapeDtypeStruct((_B, _F), jnp.float32),
        compiler_params=pltpu.CompilerParams(
            dimension_semantics=("parallel",),
        ),
    )(y, mu, sg)


def _rotl_i32(v, r):
    return lax.shift_left(v, np.int32(r)) | lax.shift_right_logical(
        v, np.int32(32 - r))


def _threefry_xored_i32(c1):
    ks2 = np.int32(0x1BD11BDB)
    x1 = c1
    x0 = x1
    x1 = _rotl_i32(x1, 13) ^ x0
    for r in (15, 26, 6):
        x0 = x0 + x1
        x1 = _rotl_i32(x1, r)
        x1 = x0 ^ x1
    x0 = x0 + np.int32(1)
    x1 = x1 + (ks2 + np.int32(1))
    rot_groups = (
        ((17, 29, 16, 24), ks2, np.int32(2)),
        ((13, 15, 26, 6), None, np.int32(4)),
        ((17, 29, 16, 24), np.int32(1), ks2 + np.int32(4)),
        ((13, 15, 26, 6), ks2, np.int32(5)),
    )
    for rots, k0inj, k1inj in rot_groups:
        for r in rots:
            x0 = x0 + x1
            x1 = _rotl_i32(x1, r)
            x1 = x0 ^ x1
        if k0inj is not None:
            x0 = x0 + k0inj
        x1 = x1 + k1inj
    return x0 ^ x1


def _sc_bits_to_normal(bits):
    fb = lax.bitcast_convert_type(
        lax.shift_right_logical(bits, np.int32(9)) | np.int32(0x40000000),
        jnp.float32)
    x = jnp.maximum(_LO, fb - np.float32(3.0))
    u = np.float32(1.0) - x * x
    bu = lax.bitcast_convert_type(u, jnp.int32)
    e_f = lax.convert_element_type(
        lax.shift_right_logical(bu, np.int32(23)), jnp.float32)
    mf = lax.bitcast_convert_type(
        (bu & np.int32(0x007FFFFF)) | np.int32(0x3F800000), jnp.float32)
    q = _MLOG_COEF[0]
    for c in _MLOG_COEF[1:]:
        q = c + q * mf
    w = e_f * _NEG_LN2 + q
    p = _ERFINV_W_COEF[0]
    for c in _ERFINV_W_COEF[1:]:
        p = c + p * w
    return p * x


_SC_ROWS_W = _SC_ROWS // _NW


def _sc_noise_body(y_hbm, mu_hbm, sg_hbm, out_hbm, ybuf, mu_v, scale_v):
    wid = lax.axis_index("s") * 2 + lax.axis_index("c")
    row0 = wid * np.int32(_SC_ROWS_W)
    pltpu.sync_copy(y_hbm.at[pl.ds(row0, _SC_ROWS_W)], ybuf)
    pltpu.sync_copy(mu_hbm, mu_v)
    pltpu.sync_copy(sg_hbm, scale_v)

    def exp_body(k, carry):
        sl = pl.ds(k * 16, 16)
        scale_v[sl] = jnp.exp(scale_v[sl])
        return carry

    lax.fori_loop(0, _F // 16, exp_body, np.int32(0))

    lane = lax.iota(jnp.int32, 16)
    ebase = row0 * np.int32(_F)
    ncol = np.int32(_F // 16)

    def body(i, carry):
        c1 = lane + (ebase + i * np.int32(16) + np.int32(1))
        z = _sc_bits_to_normal(_threefry_xored_i32(c1))
        r = lax.div(i, ncol)
        col0 = lax.rem(i, ncol) * np.int32(16)
        csl = pl.ds(col0, 16)
        ybuf[r, csl] = ybuf[r, csl] + (z * scale_v[csl] + mu_v[csl])
        return carry

    lax.fori_loop(0, _SC_CHUNKS, body, np.int32(0))
    pltpu.sync_copy(ybuf, out_hbm.at[pl.ds(row0, _SC_ROWS_W)])


def _sc_call(y, mu_flat, sg_flat):
    mesh = plsc.VectorSubcoreMesh(core_axis_name="c", subcore_axis_name="s")
    run = functools.partial(
        pl.kernel,
        out_type=jax.ShapeDtypeStruct((_SC_ROWS, _F), jnp.float32),
        mesh=mesh,
        scratch_types=[
            pltpu.VMEM((_SC_ROWS_W, _F), jnp.float32),
            pltpu.VMEM((_F,), jnp.float32),
            pltpu.VMEM((_F,), jnp.float32),
        ],
    )(_sc_noise_body)
    return run(y, mu_flat, sg_flat)


def kernel(x, y, mean, sigma):
    del x
    out_sc = _sc_call(y, mean.reshape(_F), sigma.reshape(_F))
    out_tc = _tc_call(y, mean.reshape(1, _F), sigma.reshape(1, _F))
    return lax.dynamic_update_slice(out_tc, out_sc, (0, 0))
